# baseline (speedup 1.0000x reference)
import functools

import jax
import jax.numpy as jnp
from jax import lax
from jax.experimental import pallas as pl
from jax.experimental.pallas import tpu as pltpu
from jax.experimental.pallas import tpu_sc as plsc

FP = 16
RB = 32
NB = 3


def _sc_gather_body(tpad_hbm, idx_hbm, wcat_hbm, tgt_hbm, e_out, wt_out,
                    idx_v, rows_v, tidx_v, trows_v, sem_e, sem_t,
                    *, nc, per):
    wid = lax.axis_index("s") * nc + lax.axis_index("c")
    base = wid * per
    pltpu.sync_copy(idx_hbm.at[pl.ds(base, per)], idx_v)
    pltpu.sync_copy(tgt_hbm.at[pl.ds(base, per)], tidx_v)
    ce = pltpu.async_copy(tpad_hbm.at[idx_v], rows_v, sem_e)
    ct = pltpu.async_copy(wcat_hbm.at[tidx_v], trows_v, sem_t)
    ce.wait()
    ct.wait()
    pltpu.sync_copy(rows_v, e_out.at[pl.ds(base, per)])
    pltpu.sync_copy(trows_v, wt_out.at[pl.ds(base, per)])


def _tc_body(e_ref, wT_ref, wt_ref, out_hbm, loss_ref, buf, acc_ref, sems,
             *, n_tok):
    i = pl.program_id(0)
    ni = pl.num_programs(0)
    slot = lax.rem(i, NB)

    @pl.when(i >= NB)
    def _wait_oldest():
        pltpu.make_async_copy(
            buf.at[slot], out_hbm.at[pl.ds((i - NB) * RB, RB), :],
            sems.at[slot]).wait()

    e = e_ref[...]
    logits = lax.dot_general(e, wT_ref[...], (((1,), (0,)), ((), ())),
                             preferred_element_type=jnp.float32)
    buf[slot] = logits
    pltpu.make_async_copy(
        buf.at[slot], out_hbm.at[pl.ds(i * RB, RB), :], sems.at[slot]).start()

    bm = jnp.max(logits, axis=1, keepdims=True)
    s = jnp.sum(jnp.exp(logits - bm), axis=1, keepdims=True)
    lse = bm + jnp.log(s)
    tgt = jnp.sum(e * wt_ref[...], axis=1, keepdims=True)
    part = jnp.sum(lse - tgt)
    prev = jnp.where(i == 0, 0.0, acc_ref[0])
    acc_ref[0] = prev + part

    @pl.when(i == ni - 1)
    def _fin():
        for k in range(NB):
            pltpu.make_async_copy(
                buf.at[k], out_hbm.at[pl.ds(0, RB), :], sems.at[k]).wait()
        loss_ref[0, 0] = acc_ref[0] / n_tok


def kernel(idx, targets, token_table, pos_table, W, b):
    del pos_table
    V, F = token_table.shape
    Bb, Tt = idx.shape
    N = Bb * Tt
    idx_flat = idx.reshape(N).astype(jnp.int32)
    tgt_flat = targets.reshape(N).astype(jnp.int32)

    zpad = jnp.zeros((V, FP - F - 1), jnp.float32)
    tpad = jnp.concatenate(
        [token_table, jnp.ones((V, 1), jnp.float32), zpad], axis=1)
    wcat = jnp.concatenate([W, b[:, None], zpad], axis=1)

    info = plsc.get_sparse_core_info()
    nw = info.num_cores * info.num_subcores
    per = N // nw

    sc = pl.kernel(
        functools.partial(_sc_gather_body, nc=info.num_cores, per=per),
        mesh=plsc.VectorSubcoreMesh(core_axis_name="c", subcore_axis_name="s"),
        out_type=[jax.ShapeDtypeStruct((N, FP), jnp.float32),
                  jax.ShapeDtypeStruct((N, FP), jnp.float32)],
        scratch_types=[pltpu.VMEM((per,), jnp.int32),
                       pltpu.VMEM((per, FP), jnp.float32),
                       pltpu.VMEM((per,), jnp.int32),
                       pltpu.VMEM((per, FP), jnp.float32),
                       pltpu.SemaphoreType.DMA,
                       pltpu.SemaphoreType.DMA],
        compiler_params=pltpu.CompilerParams(use_tc_tiling_on_sc=False),
    )
    epad, wt = sc(tpad, idx_flat, wcat, tgt_flat)

    ni = N // RB
    wcat_t = wcat.T

    logits, loss = pl.pallas_call(
        functools.partial(_tc_body, n_tok=N),
        grid=(ni,),
        in_specs=[
            pl.BlockSpec((RB, FP), lambda i: (i, 0)),
            pl.BlockSpec((FP, V), lambda i: (0, 0)),
            pl.BlockSpec((RB, FP), lambda i: (i, 0)),
        ],
        out_specs=[
            pl.BlockSpec(memory_space=pl.ANY),
            pl.BlockSpec((1, 1), lambda i: (0, 0), memory_space=pltpu.SMEM),
        ],
        out_shape=[
            jax.ShapeDtypeStruct((N, V), jnp.float32),
            jax.ShapeDtypeStruct((1, 1), jnp.float32),
        ],
        scratch_shapes=[
            pltpu.VMEM((NB, RB, V), jnp.float32),
            pltpu.SMEM((1,), jnp.float32),
            pltpu.SemaphoreType.DMA((NB,)),
        ],
        compiler_params=pltpu.CompilerParams(
            dimension_semantics=("arbitrary",)),
    )(epad, wcat_t, wt)

    return logits, loss[0, 0]


# speedup vs baseline: 1.0201x; 1.0201x over previous
import functools

import jax
import jax.numpy as jnp
from jax import lax
from jax.experimental import pallas as pl
from jax.experimental.pallas import tpu as pltpu
from jax.experimental.pallas import tpu_sc as plsc

FP = 16
RB = 64


def _sc_gather_body(tpad_hbm, idx_hbm, wcat_hbm, tgt_hbm, e_out, wt_out,
                    idx_v, rows_v, tidx_v, trows_v, sem_e, sem_t,
                    *, nc, per):
    wid = lax.axis_index("s") * nc + lax.axis_index("c")
    base = wid * per
    pltpu.sync_copy(idx_hbm.at[pl.ds(base, per)], idx_v)
    pltpu.sync_copy(tgt_hbm.at[pl.ds(base, per)], tidx_v)
    ce = pltpu.async_copy(tpad_hbm.at[idx_v], rows_v, sem_e)
    ct = pltpu.async_copy(wcat_hbm.at[tidx_v], trows_v, sem_t)
    ce.wait()
    ct.wait()
    pltpu.sync_copy(rows_v, e_out.at[pl.ds(base, per)])
    pltpu.sync_copy(trows_v, wt_out.at[pl.ds(base, per)])


def _tc_body(e_ref, wT_ref, wt_ref, out_ref, loss_ref, acc_ref, *, n_tok):
    i = pl.program_id(0)
    ni = pl.num_programs(0)
    e = e_ref[...]
    logits = lax.dot_general(e, wT_ref[...], (((1,), (0,)), ((), ())),
                             preferred_element_type=jnp.float32)
    out_ref[...] = logits
    bm = jnp.max(logits, axis=1, keepdims=True)
    s = jnp.sum(jnp.exp(logits - bm), axis=1, keepdims=True)
    lse = bm + jnp.log(s)
    tgt = jnp.sum(e * wt_ref[...], axis=1, keepdims=True)
    part = jnp.sum(lse - tgt)
    prev = jnp.where(i == 0, 0.0, acc_ref[0])
    acc_ref[0] = prev + part

    @pl.when(i == ni - 1)
    def _fin():
        loss_ref[0, 0] = acc_ref[0] / n_tok


def kernel(idx, targets, token_table, pos_table, W, b):
    del pos_table
    V, F = token_table.shape
    Bb, Tt = idx.shape
    N = Bb * Tt
    idx_flat = idx.reshape(N).astype(jnp.int32)
    tgt_flat = targets.reshape(N).astype(jnp.int32)

    zpad = jnp.zeros((V, FP - F - 1), jnp.float32)
    tpad = jnp.concatenate(
        [token_table, jnp.ones((V, 1), jnp.float32), zpad], axis=1)
    wcat = jnp.concatenate([W, b[:, None], zpad], axis=1)

    info = plsc.get_sparse_core_info()
    nw = info.num_cores * info.num_subcores
    per = N // nw

    sc = pl.kernel(
        functools.partial(_sc_gather_body, nc=info.num_cores, per=per),
        mesh=plsc.VectorSubcoreMesh(core_axis_name="c", subcore_axis_name="s"),
        out_type=[jax.ShapeDtypeStruct((N, FP), jnp.float32),
                  jax.ShapeDtypeStruct((N, FP), jnp.float32)],
        scratch_types=[pltpu.VMEM((per,), jnp.int32),
                       pltpu.VMEM((per, FP), jnp.float32),
                       pltpu.VMEM((per,), jnp.int32),
                       pltpu.VMEM((per, FP), jnp.float32),
                       pltpu.SemaphoreType.DMA,
                       pltpu.SemaphoreType.DMA],
        compiler_params=pltpu.CompilerParams(use_tc_tiling_on_sc=False),
    )
    epad, wt = sc(tpad, idx_flat, wcat, tgt_flat)

    ni = N // RB
    wcat_t = wcat.T

    logits, loss = pl.pallas_call(
        functools.partial(_tc_body, n_tok=N),
        grid=(ni,),
        in_specs=[
            pl.BlockSpec((RB, FP), lambda i: (i, 0)),
            pl.BlockSpec((FP, V), lambda i: (0, 0)),
            pl.BlockSpec((RB, FP), lambda i: (i, 0)),
        ],
        out_specs=[
            pl.BlockSpec((RB, V), lambda i: (i, 0)),
            pl.BlockSpec((1, 1), lambda i: (0, 0), memory_space=pltpu.SMEM),
        ],
        out_shape=[
            jax.ShapeDtypeStruct((N, V), jnp.float32),
            jax.ShapeDtypeStruct((1, 1), jnp.float32),
        ],
        scratch_shapes=[
            pltpu.SMEM((1,), jnp.float32),
        ],
        compiler_params=pltpu.CompilerParams(
            dimension_semantics=("arbitrary",)),
    )(epad, wcat_t, wt)

    return logits, loss[0, 0]
